# baseline (speedup 1.0000x reference)
import jax
import jax.numpy as jnp
from jax import lax
from jax.experimental import pallas as pl

B, D, E = 4096, 64, 8
S1, S2 = 32, 8
BLK = 2048

_EW1_R = 0
_EW2_R = 512
_SW1_R = 1024
_SW2_R = 1088
_UW1_R = 1152
_UW2_R = 1216
_EB2_R = 1280
_VEC_R = 1288
_ROWS = 1296

_DN_T = (((1,), (1,)), ((), ()))


def _dot_t(a, w):
    return lax.dot_general(a, w, _DN_T, preferred_element_type=jnp.float32)


def _moe_kernel(x_ref, ue_ref, wb_ref, b1cat_ref, out_ref):
    xb = x_ref[...]
    ue = ue_ref[...]

    vecs = wb_ref[_VEC_R:_VEC_R + 8, :]
    sb1 = vecs[0:1, :]
    sb2 = vecs[1:2, :]
    ub1 = vecs[2:3, :]
    ub2 = vecs[3:4, :]

    sw1 = wb_ref[_SW1_R:_SW1_R + D, :]
    sw2 = wb_ref[_SW2_R:_SW2_R + D, :]
    h = jnp.maximum(_dot_t(ue, sw1) + sb1, 0.0)
    logits = _dot_t(h, sw2) + sb2
    routes = jnp.argmax(logits, axis=-1).reshape(BLK, 1)

    eidx = lax.broadcasted_iota(jnp.int32, (BLK, E), 1)
    onehot = (eidx == routes).astype(jnp.float32)
    colidx = lax.broadcasted_iota(jnp.int32, (BLK, E * D), 1) // D
    maskfull = (colidx == routes).astype(jnp.float32)

    w1r = wb_ref[_EW1_R:_EW1_R + E * D, :]
    h1 = jnp.maximum(_dot_t(xb, w1r) + b1cat_ref[...], 0.0)
    h1m = h1 * maskfull

    w2stack = jnp.transpose(
        wb_ref[_EW2_R:_EW2_R + E * D, :].reshape(E, D, D),
        (0, 2, 1)).reshape(E * D, D)
    eb2 = wb_ref[_EB2_R:_EB2_R + E, :]
    out = (jnp.dot(h1m, w2stack, preferred_element_type=jnp.float32)
           + jnp.dot(onehot, eb2, preferred_element_type=jnp.float32))

    uw1 = wb_ref[_UW1_R:_UW1_R + D, :]
    uw2 = wb_ref[_UW2_R:_UW2_R + D, :]
    uh = jnp.maximum(_dot_t(xb, uw1) + ub1, 0.0)
    out = out + _dot_t(uh, uw2) + ub2

    out_ref[...] = out


@jax.jit
def kernel(x, user_embedding, SW1, Sb1, SW2, Sb2, EW1, Eb1, EW2, Eb2,
           UW1, Ub1, UW2, Ub2):
    f = jnp.float32
    z = lambda *s: jnp.zeros(s, f)
    neg = jnp.full((D - S2,), -1e30, f)
    vec_rows = jnp.concatenate([
        Sb1, z(D - S1), Sb2, neg, Ub1, Ub2]).reshape(4, D)
    blob = jnp.concatenate([
        EW1.reshape(E * D, D),
        EW2.reshape(E * D, D),
        jnp.pad(SW1, ((0, D - S1), (0, 0))),
        jnp.pad(SW2, ((0, D - S2), (0, D - S1))),
        UW1,
        UW2,
        Eb2,
        vec_rows,
        z(4, D),
    ], axis=0)

    tok = lambda i: (i, 0)
    const = lambda i: (0, 0)
    out = pl.pallas_call(
        _moe_kernel,
        grid=(B // BLK,),
        in_specs=[
            pl.BlockSpec((BLK, D), tok),
            pl.BlockSpec((BLK, D), tok),
            pl.BlockSpec((_ROWS, D), const),
            pl.BlockSpec((1, E * D), const),
        ],
        out_specs=pl.BlockSpec((BLK, D), tok),
        out_shape=jax.ShapeDtypeStruct((B, D), jnp.float32),
    )(x, user_embedding, blob, Eb1.reshape(1, E * D))
    return out


# speedup vs baseline: 1.6245x; 1.6245x over previous
import jax
import jax.numpy as jnp
from jax import lax
from jax.experimental import pallas as pl

B, D, E = 4096, 64, 8
S1, S2 = 32, 8
BLK = 2048

_DN_T = (((1,), (1,)), ((), ()))


def _dot_t(a, w):
    return lax.dot_general(a, w, _DN_T, preferred_element_type=jnp.float32)


def _moe_kernel(x_ref, ue_ref, sw1_ref, sb1_ref, sw2_ref, sb2_ref,
                w1r_ref, b1cat_ref, w2r_ref, eb2_ref,
                uw1_ref, ub1_ref, uw2_ref, ub2_ref, out_ref):
    xb = x_ref[...]
    ue = ue_ref[...]

    h = jnp.maximum(_dot_t(ue, sw1_ref[...]) + sb1_ref[...], 0.0)
    logits = _dot_t(h, sw2_ref[...]) + sb2_ref[...]
    routes = jnp.argmax(logits, axis=-1).reshape(BLK, 1)

    eidx = lax.broadcasted_iota(jnp.int32, (BLK, E), 1)
    onehot = (eidx == routes).astype(jnp.float32)
    colidx = lax.broadcasted_iota(jnp.int32, (BLK, E * D), 1) // D

    h1 = jnp.maximum(_dot_t(xb, w1r_ref[...]) + b1cat_ref[...], 0.0)
    h1m = jnp.where(colidx == routes, h1, 0.0)

    w2stack = jnp.transpose(w2r_ref[...].reshape(E, D, D),
                            (0, 2, 1)).reshape(E * D, D)
    out = (jnp.dot(h1m, w2stack, preferred_element_type=jnp.float32)
           + jnp.dot(onehot, eb2_ref[...], preferred_element_type=jnp.float32))

    uh = jnp.maximum(_dot_t(xb, uw1_ref[...]) + ub1_ref[...], 0.0)
    out = out + _dot_t(uh, uw2_ref[...]) + ub2_ref[...]

    out_ref[...] = out


@jax.jit
def kernel(x, user_embedding, SW1, Sb1, SW2, Sb2, EW1, Eb1, EW2, Eb2,
           UW1, Ub1, UW2, Ub2):
    w1r = EW1.reshape(E * D, D)
    b1cat = Eb1.reshape(1, E * D)
    w2r = EW2.reshape(E * D, D)

    tok = lambda i: (i, 0)
    const = lambda i: (0, 0)
    out = pl.pallas_call(
        _moe_kernel,
        grid=(B // BLK,),
        in_specs=[
            pl.BlockSpec((BLK, D), tok),
            pl.BlockSpec((BLK, D), tok),
            pl.BlockSpec((S1, D), const),
            pl.BlockSpec((1, S1), const),
            pl.BlockSpec((S2, S1), const),
            pl.BlockSpec((1, S2), const),
            pl.BlockSpec((E * D, D), const),
            pl.BlockSpec((1, E * D), const),
            pl.BlockSpec((E * D, D), const),
            pl.BlockSpec((E, D), const),
            pl.BlockSpec((D, D), const),
            pl.BlockSpec((1, D), const),
            pl.BlockSpec((D, D), const),
            pl.BlockSpec((1, D), const),
        ],
        out_specs=pl.BlockSpec((BLK, D), tok),
        out_shape=jax.ShapeDtypeStruct((B, D), jnp.float32),
    )(x, user_embedding, SW1, Sb1.reshape(1, S1), SW2,
      Sb2.reshape(1, S2), w1r, b1cat, w2r, Eb2,
      UW1, Ub1.reshape(1, D), UW2, Ub2.reshape(1, D))
    return out
